# baseline (speedup 1.0000x reference)
import functools

import jax
import jax.numpy as jnp
from jax import lax
from jax.experimental import pallas as pl
from jax.experimental.pallas import tpu as pltpu
from jax.experimental.pallas import tpu_sc as plsc

N = 10000
D = 128
DC = 64
C = 40
NSC = 2
NTILES = 16
EDGE_BLOCKS = NSC * NTILES
CHUNK = 192
CA = 38
CB = 67
TOTAL_CHUNKS = NTILES * (CA + CB)
E_PAD = TOTAL_CHUNKS * CHUNK
ACC_ROWS = N + 16
ZROWS = ACC_ROWS // NTILES
OROWS = N // NTILES


def _make_msgpass(d):
  mesh = plsc.VectorSubcoreMesh(core_axis_name="c", subcore_axis_name="s")

  @functools.partial(
      pl.kernel,
      out_type=jax.ShapeDtypeStruct((NSC, N, d), jnp.float32),
      mesh=mesh,
      scratch_types=[
          pltpu.VMEM((CB, CHUNK), jnp.int32),
          pltpu.VMEM((CB, CHUNK), jnp.int32),
          pltpu.VMEM((CHUNK, d), jnp.float32),
          pltpu.VMEM_SHARED((ACC_ROWS, d), jnp.float32),
          pltpu.SemaphoreType.DMA,
      ],
      compiler_params=pltpu.CompilerParams(use_tc_tiling_on_sc=False),
  )
  def msgpass(src_hbm, dst_hbm, zeros_hbm, y_hbm, out_hbm,
              idx_s, idx_d, rows, acc, sem):
    c = lax.axis_index("c")
    s = lax.axis_index("s")

    @pl.when(c == 0)
    def _():
      pltpu.sync_copy(src_hbm.at[pl.ds(s * CA, CA)], idx_s.at[pl.ds(0, CA)])
      pltpu.sync_copy(dst_hbm.at[pl.ds(s * CA, CA)], idx_d.at[pl.ds(0, CA)])

    @pl.when(c == 1)
    def _():
      pltpu.sync_copy(src_hbm.at[pl.ds(NTILES * CA + s * CB, CB)], idx_s)
      pltpu.sync_copy(dst_hbm.at[pl.ds(NTILES * CA + s * CB, CB)], idx_d)

    pltpu.sync_copy(zeros_hbm, acc.at[pl.ds(s * ZROWS, ZROWS)])
    plsc.subcore_barrier()
    n = jnp.where(c == 0, CA, CB)

    @pl.loop(0, n)
    def _(j):
      pltpu.async_copy(y_hbm.at[idx_s.at[j]], rows, sem).wait()
      pltpu.sync_copy(rows, acc.at[idx_d.at[j]], add=True)

    plsc.subcore_barrier()
    pltpu.sync_copy(acc.at[pl.ds(s * OROWS, OROWS)],
                    out_hbm.at[c, pl.ds(s * OROWS, OROWS)])

  return msgpass


_R = 1000


def _tc_fuse1(a0p, W0, b0, Wi0, bi0):
  def body(a_ref, w_ref, b_ref, wi_ref, bi_ref, h_ref, ie_ref):
    a = a_ref[0] + a_ref[1]
    h = jnp.maximum(
        jnp.dot(a, w_ref[...], preferred_element_type=jnp.float32) + b_ref[...],
        0.0)
    h_ref[...] = h
    ie_ref[...] = (
        jnp.dot(h, wi_ref[...], preferred_element_type=jnp.float32) + bi_ref[...])

  return pl.pallas_call(
      body,
      grid=(N // _R,),
      in_specs=[
          pl.BlockSpec((NSC, _R, D), lambda i: (0, i, 0)),
          pl.BlockSpec((D, D), lambda i: (0, 0)),
          pl.BlockSpec((1, D), lambda i: (0, 0)),
          pl.BlockSpec((D, C), lambda i: (0, 0)),
          pl.BlockSpec((1, C), lambda i: (0, 0)),
      ],
      out_specs=[
          pl.BlockSpec((_R, D), lambda i: (i, 0)),
          pl.BlockSpec((_R, C), lambda i: (i, 0)),
      ],
      out_shape=[
          jax.ShapeDtypeStruct((N, D), jnp.float32),
          jax.ShapeDtypeStruct((N, C), jnp.float32),
      ],
  )(a0p, W0, b0, Wi0, bi0)


def _tc_fuse2(a1p, W1, b1, Wi1, bi1, W2p, b2, ie0):
  def body(a_ref, w_ref, b_ref, wi_ref, bi_ref, w2_ref, b2_ref, ie_ref,
           ie2_ref, y2_ref):
    a = a_ref[0] + a_ref[1]
    h = jnp.maximum(
        jnp.dot(a, w_ref[...], preferred_element_type=jnp.float32) + b_ref[...],
        0.0)
    ie2_ref[...] = (
        ie_ref[...]
        + jnp.dot(h, wi_ref[...], preferred_element_type=jnp.float32)
        + bi_ref[...] + b2_ref[...])
    y2_ref[...] = jnp.dot(h, w2_ref[...], preferred_element_type=jnp.float32)

  return pl.pallas_call(
      body,
      grid=(N // _R,),
      in_specs=[
          pl.BlockSpec((NSC, _R, D), lambda i: (0, i, 0)),
          pl.BlockSpec((D, D), lambda i: (0, 0)),
          pl.BlockSpec((1, D), lambda i: (0, 0)),
          pl.BlockSpec((D, C), lambda i: (0, 0)),
          pl.BlockSpec((1, C), lambda i: (0, 0)),
          pl.BlockSpec((D, DC), lambda i: (0, 0)),
          pl.BlockSpec((1, C), lambda i: (0, 0)),
          pl.BlockSpec((_R, C), lambda i: (i, 0)),
      ],
      out_specs=[
          pl.BlockSpec((_R, C), lambda i: (i, 0)),
          pl.BlockSpec((_R, DC), lambda i: (i, 0)),
      ],
      out_shape=[
          jax.ShapeDtypeStruct((N, C), jnp.float32),
          jax.ShapeDtypeStruct((N, DC), jnp.float32),
      ],
  )(a1p, W1, b1, Wi1, bi1, W2p, b2, ie0)


def _tc_fuse3(a2p, ie2):
  def body(a_ref, ie_ref, o_ref):
    o_ref[...] = a_ref[0][:, :C] + a_ref[1][:, :C] + ie_ref[...]

  return pl.pallas_call(
      body,
      grid=(N // _R,),
      in_specs=[
          pl.BlockSpec((NSC, _R, DC), lambda i: (0, i, 0)),
          pl.BlockSpec((_R, C), lambda i: (i, 0)),
      ],
      out_specs=pl.BlockSpec((_R, C), lambda i: (i, 0)),
      out_shape=jax.ShapeDtypeStruct((N, C), jnp.float32),
  )(a2p, ie2)


def kernel(graph, features, W0, b0, W1, b1, W2, b2, Wi0, bi0, Wi1, bi1):
  src, dst = graph[0], graph[1]
  pad = E_PAD - src.shape[0]
  srcp = jnp.concatenate([src, jnp.zeros((pad,), jnp.int32)]).reshape(
      TOTAL_CHUNKS, CHUNK)
  dstp = jnp.concatenate([dst, jnp.full((pad,), N, jnp.int32)]).reshape(
      TOTAL_CHUNKS, CHUNK)
  zeros128 = jnp.zeros((ZROWS, D), jnp.float32)
  zeros64 = jnp.zeros((ZROWS, DC), jnp.float32)

  mp128 = _make_msgpass(D)
  mp64 = _make_msgpass(DC)

  a0p = mp128(srcp, dstp, zeros128, features)
  h1, ie0 = _tc_fuse1(a0p, W0, b0.reshape(1, D), Wi0, bi0.reshape(1, C))
  a1p = mp128(srcp, dstp, zeros128, h1)
  W2p = jnp.pad(W2, ((0, 0), (0, DC - C)))
  ie2, y2 = _tc_fuse2(a1p, W1, b1.reshape(1, D), Wi1, bi1.reshape(1, C),
                      W2p, b2.reshape(1, C), ie0)
  a2p = mp64(srcp, dstp, zeros64, y2)
  return _tc_fuse3(a2p, ie2)


# speedup vs baseline: 1.2501x; 1.2501x over previous
import functools

import jax
import jax.numpy as jnp
from jax import lax
from jax.experimental import pallas as pl
from jax.experimental.pallas import tpu as pltpu
from jax.experimental.pallas import tpu_sc as plsc

N = 10000
D = 128
DC = 64
C = 40
NSC = 2
NTILES = 16
EDGE_BLOCKS = NSC * NTILES
CHUNK = 192
CA = 67
CB = 38
CMAX = max(CA, CB)
TOTAL_CHUNKS = NTILES * (CA + CB)
E_PAD = TOTAL_CHUNKS * CHUNK
ACC_ROWS = N + 16
ZROWS = ACC_ROWS // NTILES
OROWS = N // NTILES


def _make_msgpass(d):
  mesh = plsc.VectorSubcoreMesh(core_axis_name="c", subcore_axis_name="s")

  @functools.partial(
      pl.kernel,
      out_type=jax.ShapeDtypeStruct((NSC, N, d), jnp.float32),
      mesh=mesh,
      scratch_types=[
          pltpu.VMEM((CMAX, CHUNK), jnp.int32),
          pltpu.VMEM((CMAX, CHUNK), jnp.int32),
          pltpu.VMEM((CHUNK, d), jnp.float32),
          pltpu.VMEM_SHARED((ACC_ROWS, d), jnp.float32),
          pltpu.SemaphoreType.DMA,
      ],
      compiler_params=pltpu.CompilerParams(use_tc_tiling_on_sc=False),
  )
  def msgpass(src_hbm, dst_hbm, zeros_hbm, y_hbm, out_hbm,
              idx_s, idx_d, rows, acc, sem):
    c = lax.axis_index("c")
    s = lax.axis_index("s")

    @pl.when(c == 0)
    def _():
      pltpu.sync_copy(src_hbm.at[pl.ds(s * CA, CA)], idx_s.at[pl.ds(0, CA)])
      pltpu.sync_copy(dst_hbm.at[pl.ds(s * CA, CA)], idx_d.at[pl.ds(0, CA)])

    @pl.when(c == 1)
    def _():
      pltpu.sync_copy(src_hbm.at[pl.ds(NTILES * CA + s * CB, CB)],
                      idx_s.at[pl.ds(0, CB)])
      pltpu.sync_copy(dst_hbm.at[pl.ds(NTILES * CA + s * CB, CB)],
                      idx_d.at[pl.ds(0, CB)])

    pltpu.sync_copy(zeros_hbm, acc.at[pl.ds(s * ZROWS, ZROWS)])
    plsc.subcore_barrier()
    n = jnp.where(c == 0, CA, CB)

    @pl.loop(0, n)
    def _(j):
      pltpu.async_copy(y_hbm.at[idx_s.at[j]], rows, sem).wait()
      pltpu.sync_copy(rows, acc.at[idx_d.at[j]], add=True)

    plsc.subcore_barrier()
    pltpu.sync_copy(acc.at[pl.ds(s * OROWS, OROWS)],
                    out_hbm.at[c, pl.ds(s * OROWS, OROWS)])

  return msgpass


_R = 1000


def _tc_fuse1(a0p, W0, b0, Wi0, bi0):
  def body(a_ref, w_ref, b_ref, wi_ref, bi_ref, h_ref, ie_ref):
    a = a_ref[0] + a_ref[1]
    h = jnp.maximum(
        jnp.dot(a, w_ref[...], preferred_element_type=jnp.float32) + b_ref[...],
        0.0)
    h_ref[...] = h
    ie_ref[...] = (
        jnp.dot(h, wi_ref[...], preferred_element_type=jnp.float32) + bi_ref[...])

  return pl.pallas_call(
      body,
      grid=(N // _R,),
      in_specs=[
          pl.BlockSpec((NSC, _R, D), lambda i: (0, i, 0)),
          pl.BlockSpec((D, D), lambda i: (0, 0)),
          pl.BlockSpec((1, D), lambda i: (0, 0)),
          pl.BlockSpec((D, C), lambda i: (0, 0)),
          pl.BlockSpec((1, C), lambda i: (0, 0)),
      ],
      out_specs=[
          pl.BlockSpec((_R, D), lambda i: (i, 0)),
          pl.BlockSpec((_R, C), lambda i: (i, 0)),
      ],
      out_shape=[
          jax.ShapeDtypeStruct((N, D), jnp.float32),
          jax.ShapeDtypeStruct((N, C), jnp.float32),
      ],
  )(a0p, W0, b0, Wi0, bi0)


def _tc_fuse2(a1p, W1, b1, Wi1, bi1, W2p, b2, ie0):
  def body(a_ref, w_ref, b_ref, wi_ref, bi_ref, w2_ref, b2_ref, ie_ref,
           ie2_ref, y2_ref):
    a = a_ref[0] + a_ref[1]
    h = jnp.maximum(
        jnp.dot(a, w_ref[...], preferred_element_type=jnp.float32) + b_ref[...],
        0.0)
    ie2_ref[...] = (
        ie_ref[...]
        + jnp.dot(h, wi_ref[...], preferred_element_type=jnp.float32)
        + bi_ref[...] + b2_ref[...])
    y2_ref[...] = jnp.dot(h, w2_ref[...], preferred_element_type=jnp.float32)

  return pl.pallas_call(
      body,
      grid=(N // _R,),
      in_specs=[
          pl.BlockSpec((NSC, _R, D), lambda i: (0, i, 0)),
          pl.BlockSpec((D, D), lambda i: (0, 0)),
          pl.BlockSpec((1, D), lambda i: (0, 0)),
          pl.BlockSpec((D, C), lambda i: (0, 0)),
          pl.BlockSpec((1, C), lambda i: (0, 0)),
          pl.BlockSpec((D, DC), lambda i: (0, 0)),
          pl.BlockSpec((1, C), lambda i: (0, 0)),
          pl.BlockSpec((_R, C), lambda i: (i, 0)),
      ],
      out_specs=[
          pl.BlockSpec((_R, C), lambda i: (i, 0)),
          pl.BlockSpec((_R, DC), lambda i: (i, 0)),
      ],
      out_shape=[
          jax.ShapeDtypeStruct((N, C), jnp.float32),
          jax.ShapeDtypeStruct((N, DC), jnp.float32),
      ],
  )(a1p, W1, b1, Wi1, bi1, W2p, b2, ie0)


def _tc_fuse3(a2p, ie2):
  def body(a_ref, ie_ref, o_ref):
    o_ref[...] = a_ref[0][:, :C] + a_ref[1][:, :C] + ie_ref[...]

  return pl.pallas_call(
      body,
      grid=(N // _R,),
      in_specs=[
          pl.BlockSpec((NSC, _R, DC), lambda i: (0, i, 0)),
          pl.BlockSpec((_R, C), lambda i: (i, 0)),
      ],
      out_specs=pl.BlockSpec((_R, C), lambda i: (i, 0)),
      out_shape=jax.ShapeDtypeStruct((N, C), jnp.float32),
  )(a2p, ie2)


def kernel(graph, features, W0, b0, W1, b1, W2, b2, Wi0, bi0, Wi1, bi1):
  src, dst = graph[0], graph[1]
  pad = E_PAD - src.shape[0]
  srcp = jnp.concatenate([src, jnp.zeros((pad,), jnp.int32)]).reshape(
      TOTAL_CHUNKS, CHUNK)
  dstp = jnp.concatenate([dst, jnp.full((pad,), N, jnp.int32)]).reshape(
      TOTAL_CHUNKS, CHUNK)
  zeros128 = jnp.zeros((ZROWS, D), jnp.float32)
  zeros64 = jnp.zeros((ZROWS, DC), jnp.float32)

  mp128 = _make_msgpass(D)
  mp64 = _make_msgpass(DC)

  a0p = mp128(srcp, dstp, zeros128, features)
  h1, ie0 = _tc_fuse1(a0p, W0, b0.reshape(1, D), Wi0, bi0.reshape(1, C))
  a1p = mp128(srcp, dstp, zeros128, h1)
  W2p = jnp.pad(W2, ((0, 0), (0, DC - C)))
  ie2, y2 = _tc_fuse2(a1p, W1, b1.reshape(1, D), Wi1, bi1.reshape(1, C),
                      W2p, b2.reshape(1, C), ie0)
  a2p = mp64(srcp, dstp, zeros64, y2)
  return _tc_fuse3(a2p, ie2)


# speedup vs baseline: 1.2528x; 1.0021x over previous
import functools

import jax
import jax.numpy as jnp
from jax import lax
from jax.experimental import pallas as pl
from jax.experimental.pallas import tpu as pltpu
from jax.experimental.pallas import tpu_sc as plsc

N = 10000
D = 128
DC = 48
C = 40
NSC = 2
NTILES = 16
EDGE_BLOCKS = NSC * NTILES
CHUNK = 192
CA = 63
CB = 42
CMAX = max(CA, CB)
TOTAL_CHUNKS = NTILES * (CA + CB)
E_PAD = TOTAL_CHUNKS * CHUNK
ACC_ROWS = N + 16
ZROWS = ACC_ROWS // NTILES
OROWS = N // NTILES


def _make_msgpass(d):
  mesh = plsc.VectorSubcoreMesh(core_axis_name="c", subcore_axis_name="s")

  @functools.partial(
      pl.kernel,
      out_type=jax.ShapeDtypeStruct((NSC, N, d), jnp.float32),
      mesh=mesh,
      scratch_types=[
          pltpu.VMEM((CMAX, CHUNK), jnp.int32),
          pltpu.VMEM((CMAX, CHUNK), jnp.int32),
          pltpu.VMEM((CHUNK, d), jnp.float32),
          pltpu.VMEM_SHARED((ACC_ROWS, d), jnp.float32),
          pltpu.SemaphoreType.DMA,
      ],
      compiler_params=pltpu.CompilerParams(use_tc_tiling_on_sc=False),
  )
  def msgpass(src_hbm, dst_hbm, zeros_hbm, y_hbm, out_hbm,
              idx_s, idx_d, rows, acc, sem):
    c = lax.axis_index("c")
    s = lax.axis_index("s")

    @pl.when(c == 0)
    def _():
      pltpu.sync_copy(src_hbm.at[pl.ds(s * CA, CA)], idx_s.at[pl.ds(0, CA)])
      pltpu.sync_copy(dst_hbm.at[pl.ds(s * CA, CA)], idx_d.at[pl.ds(0, CA)])

    @pl.when(c == 1)
    def _():
      pltpu.sync_copy(src_hbm.at[pl.ds(NTILES * CA + s * CB, CB)],
                      idx_s.at[pl.ds(0, CB)])
      pltpu.sync_copy(dst_hbm.at[pl.ds(NTILES * CA + s * CB, CB)],
                      idx_d.at[pl.ds(0, CB)])

    pltpu.sync_copy(zeros_hbm, acc.at[pl.ds(s * ZROWS, ZROWS)])
    plsc.subcore_barrier()
    n = jnp.where(c == 0, CA, CB)

    @pl.loop(0, n)
    def _(j):
      pltpu.async_copy(y_hbm.at[idx_s.at[j]], rows, sem).wait()
      pltpu.sync_copy(rows, acc.at[idx_d.at[j]], add=True)

    plsc.subcore_barrier()
    pltpu.sync_copy(acc.at[pl.ds(s * OROWS, OROWS)],
                    out_hbm.at[c, pl.ds(s * OROWS, OROWS)])

  return msgpass


_R = 1000


def _tc_fuse1(a0p, W0, b0, Wi0, bi0):
  def body(a_ref, w_ref, b_ref, wi_ref, bi_ref, h_ref, ie_ref):
    a = a_ref[0] + a_ref[1]
    h = jnp.maximum(
        jnp.dot(a, w_ref[...], preferred_element_type=jnp.float32) + b_ref[...],
        0.0)
    h_ref[...] = h
    ie_ref[...] = (
        jnp.dot(h, wi_ref[...], preferred_element_type=jnp.float32) + bi_ref[...])

  return pl.pallas_call(
      body,
      grid=(N // _R,),
      in_specs=[
          pl.BlockSpec((NSC, _R, D), lambda i: (0, i, 0)),
          pl.BlockSpec((D, D), lambda i: (0, 0)),
          pl.BlockSpec((1, D), lambda i: (0, 0)),
          pl.BlockSpec((D, C), lambda i: (0, 0)),
          pl.BlockSpec((1, C), lambda i: (0, 0)),
      ],
      out_specs=[
          pl.BlockSpec((_R, D), lambda i: (i, 0)),
          pl.BlockSpec((_R, C), lambda i: (i, 0)),
      ],
      out_shape=[
          jax.ShapeDtypeStruct((N, D), jnp.float32),
          jax.ShapeDtypeStruct((N, C), jnp.float32),
      ],
  )(a0p, W0, b0, Wi0, bi0)


def _tc_fuse2(a1p, W1, b1, Wi1, bi1, W2p, b2, ie0):
  def body(a_ref, w_ref, b_ref, wi_ref, bi_ref, w2_ref, b2_ref, ie_ref,
           ie2_ref, y2_ref):
    a = a_ref[0] + a_ref[1]
    h = jnp.maximum(
        jnp.dot(a, w_ref[...], preferred_element_type=jnp.float32) + b_ref[...],
        0.0)
    ie2_ref[...] = (
        ie_ref[...]
        + jnp.dot(h, wi_ref[...], preferred_element_type=jnp.float32)
        + bi_ref[...] + b2_ref[...])
    y2_ref[...] = jnp.dot(h, w2_ref[...], preferred_element_type=jnp.float32)

  return pl.pallas_call(
      body,
      grid=(N // _R,),
      in_specs=[
          pl.BlockSpec((NSC, _R, D), lambda i: (0, i, 0)),
          pl.BlockSpec((D, D), lambda i: (0, 0)),
          pl.BlockSpec((1, D), lambda i: (0, 0)),
          pl.BlockSpec((D, C), lambda i: (0, 0)),
          pl.BlockSpec((1, C), lambda i: (0, 0)),
          pl.BlockSpec((D, DC), lambda i: (0, 0)),
          pl.BlockSpec((1, C), lambda i: (0, 0)),
          pl.BlockSpec((_R, C), lambda i: (i, 0)),
      ],
      out_specs=[
          pl.BlockSpec((_R, C), lambda i: (i, 0)),
          pl.BlockSpec((_R, DC), lambda i: (i, 0)),
      ],
      out_shape=[
          jax.ShapeDtypeStruct((N, C), jnp.float32),
          jax.ShapeDtypeStruct((N, DC), jnp.float32),
      ],
  )(a1p, W1, b1, Wi1, bi1, W2p, b2, ie0)


def _tc_fuse3(a2p, ie2):
  def body(a_ref, ie_ref, o_ref):
    o_ref[...] = a_ref[0][:, :C] + a_ref[1][:, :C] + ie_ref[...]

  return pl.pallas_call(
      body,
      grid=(N // _R,),
      in_specs=[
          pl.BlockSpec((NSC, _R, DC), lambda i: (0, i, 0)),
          pl.BlockSpec((_R, C), lambda i: (i, 0)),
      ],
      out_specs=pl.BlockSpec((_R, C), lambda i: (i, 0)),
      out_shape=jax.ShapeDtypeStruct((N, C), jnp.float32),
  )(a2p, ie2)


def kernel(graph, features, W0, b0, W1, b1, W2, b2, Wi0, bi0, Wi1, bi1):
  src, dst = graph[0], graph[1]
  pad = E_PAD - src.shape[0]
  srcp = jnp.concatenate([src, jnp.zeros((pad,), jnp.int32)]).reshape(
      TOTAL_CHUNKS, CHUNK)
  dstp = jnp.concatenate([dst, jnp.full((pad,), N, jnp.int32)]).reshape(
      TOTAL_CHUNKS, CHUNK)
  zeros128 = jnp.zeros((ZROWS, D), jnp.float32)
  zeros64 = jnp.zeros((ZROWS, DC), jnp.float32)

  mp128 = _make_msgpass(D)
  mp64 = _make_msgpass(DC)

  a0p = mp128(srcp, dstp, zeros128, features)
  h1, ie0 = _tc_fuse1(a0p, W0, b0.reshape(1, D), Wi0, bi0.reshape(1, C))
  a1p = mp128(srcp, dstp, zeros128, h1)
  W2p = jnp.pad(W2, ((0, 0), (0, DC - C)))
  ie2, y2 = _tc_fuse2(a1p, W1, b1.reshape(1, D), Wi1, bi1.reshape(1, C),
                      W2p, b2.reshape(1, C), ie0)
  a2p = mp64(srcp, dstp, zeros64, y2)
  return _tc_fuse3(a2p, ie2)


# speedup vs baseline: 1.5582x; 1.2438x over previous
import functools

import jax
import jax.numpy as jnp
from jax import lax
from jax.experimental import pallas as pl
from jax.experimental.pallas import tpu as pltpu
from jax.experimental.pallas import tpu_sc as plsc

N = 10000
D = 128
DC = 48
C = 40
NSC = 2
NTILES = 16
EDGE_BLOCKS = NSC * NTILES
CHUNK = 200
CA = 60
CB = 40
CMAX = max(CA, CB)
TOTAL_CHUNKS = NTILES * (CA + CB)
E_PAD = TOTAL_CHUNKS * CHUNK
ACC_ROWS = N + 16
ZROWS = ACC_ROWS // NTILES
OROWS = N // NTILES


def _make_msgpass(d):
  mesh = plsc.VectorSubcoreMesh(core_axis_name="c", subcore_axis_name="s")

  @functools.partial(
      pl.kernel,
      out_type=jax.ShapeDtypeStruct((NSC, N, d), jnp.float32),
      mesh=mesh,
      scratch_types=[
          pltpu.VMEM((CMAX, CHUNK), jnp.int32),
          pltpu.VMEM((CMAX, CHUNK), jnp.int32),
          pltpu.VMEM((CHUNK, d), jnp.float32),
          pltpu.VMEM_SHARED((ACC_ROWS, d), jnp.float32),
          pltpu.SemaphoreType.DMA,
      ],
      compiler_params=pltpu.CompilerParams(use_tc_tiling_on_sc=False),
  )
  def msgpass(src_hbm, dst_hbm, zeros_hbm, y_hbm, out_hbm,
              idx_s, idx_d, rows, acc, sem):
    c = lax.axis_index("c")
    s = lax.axis_index("s")

    @pl.when(c == 0)
    def _():
      pltpu.sync_copy(src_hbm.at[pl.ds(s * CA, CA)], idx_s.at[pl.ds(0, CA)])
      pltpu.sync_copy(dst_hbm.at[pl.ds(s * CA, CA)], idx_d.at[pl.ds(0, CA)])

    @pl.when(c == 1)
    def _():
      pltpu.sync_copy(src_hbm.at[pl.ds(NTILES * CA + s * CB, CB)],
                      idx_s.at[pl.ds(0, CB)])
      pltpu.sync_copy(dst_hbm.at[pl.ds(NTILES * CA + s * CB, CB)],
                      idx_d.at[pl.ds(0, CB)])

    pltpu.sync_copy(zeros_hbm, acc.at[pl.ds(s * ZROWS, ZROWS)])
    plsc.subcore_barrier()
    n = jnp.where(c == 0, CA, CB)

    @pl.loop(0, n)
    def _(j):
      pltpu.async_copy(y_hbm.at[idx_s.at[j]], rows, sem).wait()
      pltpu.sync_copy(rows, acc.at[idx_d.at[j]], add=True)

    plsc.subcore_barrier()
    pltpu.sync_copy(acc.at[pl.ds(s * OROWS, OROWS)],
                    out_hbm.at[c, pl.ds(s * OROWS, OROWS)])

  return msgpass


_R = 1000


def _tc_fuse1(a0p, W0, b0, Wi0, bi0):
  def body(a_ref, w_ref, b_ref, wi_ref, bi_ref, h_ref, ie_ref):
    a = a_ref[0] + a_ref[1]
    h = jnp.maximum(
        jnp.dot(a, w_ref[...], preferred_element_type=jnp.float32) + b_ref[...],
        0.0)
    h_ref[...] = h
    ie_ref[...] = (
        jnp.dot(h, wi_ref[...], preferred_element_type=jnp.float32) + bi_ref[...])

  return pl.pallas_call(
      body,
      grid=(N // _R,),
      in_specs=[
          pl.BlockSpec((NSC, _R, D), lambda i: (0, i, 0)),
          pl.BlockSpec((D, D), lambda i: (0, 0)),
          pl.BlockSpec((1, D), lambda i: (0, 0)),
          pl.BlockSpec((D, C), lambda i: (0, 0)),
          pl.BlockSpec((1, C), lambda i: (0, 0)),
      ],
      out_specs=[
          pl.BlockSpec((_R, D), lambda i: (i, 0)),
          pl.BlockSpec((_R, C), lambda i: (i, 0)),
      ],
      out_shape=[
          jax.ShapeDtypeStruct((N, D), jnp.float32),
          jax.ShapeDtypeStruct((N, C), jnp.float32),
      ],
  )(a0p, W0, b0, Wi0, bi0)


def _tc_fuse2(a1p, W1, b1, Wi1, bi1, W2p, b2, ie0):
  def body(a_ref, w_ref, b_ref, wi_ref, bi_ref, w2_ref, b2_ref, ie_ref,
           ie2_ref, y2_ref):
    a = a_ref[0] + a_ref[1]
    h = jnp.maximum(
        jnp.dot(a, w_ref[...], preferred_element_type=jnp.float32) + b_ref[...],
        0.0)
    ie2_ref[...] = (
        ie_ref[...]
        + jnp.dot(h, wi_ref[...], preferred_element_type=jnp.float32)
        + bi_ref[...] + b2_ref[...])
    y2_ref[...] = jnp.dot(h, w2_ref[...], preferred_element_type=jnp.float32)

  return pl.pallas_call(
      body,
      grid=(N // _R,),
      in_specs=[
          pl.BlockSpec((NSC, _R, D), lambda i: (0, i, 0)),
          pl.BlockSpec((D, D), lambda i: (0, 0)),
          pl.BlockSpec((1, D), lambda i: (0, 0)),
          pl.BlockSpec((D, C), lambda i: (0, 0)),
          pl.BlockSpec((1, C), lambda i: (0, 0)),
          pl.BlockSpec((D, DC), lambda i: (0, 0)),
          pl.BlockSpec((1, C), lambda i: (0, 0)),
          pl.BlockSpec((_R, C), lambda i: (i, 0)),
      ],
      out_specs=[
          pl.BlockSpec((_R, C), lambda i: (i, 0)),
          pl.BlockSpec((_R, DC), lambda i: (i, 0)),
      ],
      out_shape=[
          jax.ShapeDtypeStruct((N, C), jnp.float32),
          jax.ShapeDtypeStruct((N, DC), jnp.float32),
      ],
  )(a1p, W1, b1, Wi1, bi1, W2p, b2, ie0)


def _tc_fuse3(a2p, ie2):
  def body(a_ref, ie_ref, o_ref):
    o_ref[...] = a_ref[0][:, :C] + a_ref[1][:, :C] + ie_ref[...]

  return pl.pallas_call(
      body,
      grid=(N // _R,),
      in_specs=[
          pl.BlockSpec((NSC, _R, DC), lambda i: (0, i, 0)),
          pl.BlockSpec((_R, C), lambda i: (i, 0)),
      ],
      out_specs=pl.BlockSpec((_R, C), lambda i: (i, 0)),
      out_shape=jax.ShapeDtypeStruct((N, C), jnp.float32),
  )(a2p, ie2)


def kernel(graph, features, W0, b0, W1, b1, W2, b2, Wi0, bi0, Wi1, bi1):
  src, dst = graph[0], graph[1]
  assert E_PAD == src.shape[0]
  srcp = src.reshape(TOTAL_CHUNKS, CHUNK)
  dstp = dst.reshape(TOTAL_CHUNKS, CHUNK)
  zeros128 = jnp.zeros((ZROWS, D), jnp.float32)
  zeros64 = jnp.zeros((ZROWS, DC), jnp.float32)

  mp128 = _make_msgpass(D)
  mp64 = _make_msgpass(DC)

  a0p = mp128(srcp, dstp, zeros128, features)
  h1, ie0 = _tc_fuse1(a0p, W0, b0.reshape(1, D), Wi0, bi0.reshape(1, C))
  a1p = mp128(srcp, dstp, zeros128, h1)
  W2p = jnp.pad(W2, ((0, 0), (0, DC - C)))
  ie2, y2 = _tc_fuse2(a1p, W1, b1.reshape(1, D), Wi1, bi1.reshape(1, C),
                      W2p, b2.reshape(1, C), ie0)
  a2p = mp64(srcp, dstp, zeros64, y2)
  return _tc_fuse3(a2p, ie2)


# speedup vs baseline: 1.7532x; 1.1252x over previous
import functools

import jax
import jax.numpy as jnp
from jax import lax
from jax.experimental import pallas as pl
from jax.experimental.pallas import tpu as pltpu
from jax.experimental.pallas import tpu_sc as plsc

N = 10000
D = 128
DC = 48
C = 40
NSC = 2
NTILES = 16
EDGE_BLOCKS = NSC * NTILES
CHUNK = 200
CA = 51
CB = 49
CMAX = max(CA, CB)
TOTAL_CHUNKS = NTILES * (CA + CB)
E_PAD = TOTAL_CHUNKS * CHUNK
ACC_ROWS = N + 16
ZROWS = ACC_ROWS // NTILES
OROWS = N // NTILES


def _make_msgpass(d):
  mesh = plsc.VectorSubcoreMesh(core_axis_name="c", subcore_axis_name="s")

  @functools.partial(
      pl.kernel,
      out_type=jax.ShapeDtypeStruct((NSC, N, d), jnp.float32),
      mesh=mesh,
      scratch_types=[
          pltpu.VMEM((CMAX, CHUNK), jnp.int32),
          pltpu.VMEM((CMAX, CHUNK), jnp.int32),
          pltpu.VMEM((CHUNK, d), jnp.float32),
          pltpu.VMEM_SHARED((ACC_ROWS, d), jnp.float32),
          pltpu.SemaphoreType.DMA,
      ],
      compiler_params=pltpu.CompilerParams(use_tc_tiling_on_sc=False),
  )
  def msgpass(src_hbm, dst_hbm, zeros_hbm, y_hbm, out_hbm,
              idx_s, idx_d, rows, acc, sem):
    c = lax.axis_index("c")
    s = lax.axis_index("s")

    @pl.when(c == 0)
    def _():
      pltpu.sync_copy(src_hbm.at[pl.ds(s * CA, CA)], idx_s.at[pl.ds(0, CA)])
      pltpu.sync_copy(dst_hbm.at[pl.ds(s * CA, CA)], idx_d.at[pl.ds(0, CA)])

    @pl.when(c == 1)
    def _():
      pltpu.sync_copy(src_hbm.at[pl.ds(NTILES * CA + s * CB, CB)],
                      idx_s.at[pl.ds(0, CB)])
      pltpu.sync_copy(dst_hbm.at[pl.ds(NTILES * CA + s * CB, CB)],
                      idx_d.at[pl.ds(0, CB)])

    pltpu.sync_copy(zeros_hbm, acc.at[pl.ds(s * ZROWS, ZROWS)])
    plsc.subcore_barrier()
    n = jnp.where(c == 0, CA, CB)

    @pl.loop(0, n)
    def _(j):
      pltpu.async_copy(y_hbm.at[idx_s.at[j]], rows, sem).wait()
      pltpu.sync_copy(rows, acc.at[idx_d.at[j]], add=True)

    plsc.subcore_barrier()
    pltpu.sync_copy(acc.at[pl.ds(s * OROWS, OROWS)],
                    out_hbm.at[c, pl.ds(s * OROWS, OROWS)])

  return msgpass


_R = 1000


def _tc_fuse1(a0p, W0, b0, Wi0, bi0):
  def body(a_ref, w_ref, b_ref, wi_ref, bi_ref, h_ref, ie_ref):
    a = a_ref[0] + a_ref[1]
    h = jnp.maximum(
        jnp.dot(a, w_ref[...], preferred_element_type=jnp.float32) + b_ref[...],
        0.0)
    h_ref[...] = h
    ie_ref[...] = (
        jnp.dot(h, wi_ref[...], preferred_element_type=jnp.float32) + bi_ref[...])

  return pl.pallas_call(
      body,
      grid=(N // _R,),
      in_specs=[
          pl.BlockSpec((NSC, _R, D), lambda i: (0, i, 0)),
          pl.BlockSpec((D, D), lambda i: (0, 0)),
          pl.BlockSpec((1, D), lambda i: (0, 0)),
          pl.BlockSpec((D, C), lambda i: (0, 0)),
          pl.BlockSpec((1, C), lambda i: (0, 0)),
      ],
      out_specs=[
          pl.BlockSpec((_R, D), lambda i: (i, 0)),
          pl.BlockSpec((_R, C), lambda i: (i, 0)),
      ],
      out_shape=[
          jax.ShapeDtypeStruct((N, D), jnp.float32),
          jax.ShapeDtypeStruct((N, C), jnp.float32),
      ],
  )(a0p, W0, b0, Wi0, bi0)


def _tc_fuse2(a1p, W1, b1, Wi1, bi1, W2p, b2, ie0):
  def body(a_ref, w_ref, b_ref, wi_ref, bi_ref, w2_ref, b2_ref, ie_ref,
           ie2_ref, y2_ref):
    a = a_ref[0] + a_ref[1]
    h = jnp.maximum(
        jnp.dot(a, w_ref[...], preferred_element_type=jnp.float32) + b_ref[...],
        0.0)
    ie2_ref[...] = (
        ie_ref[...]
        + jnp.dot(h, wi_ref[...], preferred_element_type=jnp.float32)
        + bi_ref[...] + b2_ref[...])
    y2_ref[...] = jnp.dot(h, w2_ref[...], preferred_element_type=jnp.float32)

  return pl.pallas_call(
      body,
      grid=(N // _R,),
      in_specs=[
          pl.BlockSpec((NSC, _R, D), lambda i: (0, i, 0)),
          pl.BlockSpec((D, D), lambda i: (0, 0)),
          pl.BlockSpec((1, D), lambda i: (0, 0)),
          pl.BlockSpec((D, C), lambda i: (0, 0)),
          pl.BlockSpec((1, C), lambda i: (0, 0)),
          pl.BlockSpec((D, DC), lambda i: (0, 0)),
          pl.BlockSpec((1, C), lambda i: (0, 0)),
          pl.BlockSpec((_R, C), lambda i: (i, 0)),
      ],
      out_specs=[
          pl.BlockSpec((_R, C), lambda i: (i, 0)),
          pl.BlockSpec((_R, DC), lambda i: (i, 0)),
      ],
      out_shape=[
          jax.ShapeDtypeStruct((N, C), jnp.float32),
          jax.ShapeDtypeStruct((N, DC), jnp.float32),
      ],
  )(a1p, W1, b1, Wi1, bi1, W2p, b2, ie0)


def _tc_fuse3(a2p, ie2):
  def body(a_ref, ie_ref, o_ref):
    o_ref[...] = a_ref[0][:, :C] + a_ref[1][:, :C] + ie_ref[...]

  return pl.pallas_call(
      body,
      grid=(N // _R,),
      in_specs=[
          pl.BlockSpec((NSC, _R, DC), lambda i: (0, i, 0)),
          pl.BlockSpec((_R, C), lambda i: (i, 0)),
      ],
      out_specs=pl.BlockSpec((_R, C), lambda i: (i, 0)),
      out_shape=jax.ShapeDtypeStruct((N, C), jnp.float32),
  )(a2p, ie2)


def kernel(graph, features, W0, b0, W1, b1, W2, b2, Wi0, bi0, Wi1, bi1):
  src, dst = graph[0], graph[1]
  assert E_PAD == src.shape[0]
  srcp = src.reshape(TOTAL_CHUNKS, CHUNK)
  dstp = dst.reshape(TOTAL_CHUNKS, CHUNK)
  zeros128 = jnp.zeros((ZROWS, D), jnp.float32)
  zeros64 = jnp.zeros((ZROWS, DC), jnp.float32)

  mp128 = _make_msgpass(D)
  mp64 = _make_msgpass(DC)

  a0p = mp128(srcp, dstp, zeros128, features)
  h1, ie0 = _tc_fuse1(a0p, W0, b0.reshape(1, D), Wi0, bi0.reshape(1, C))
  a1p = mp128(srcp, dstp, zeros128, h1)
  W2p = jnp.pad(W2, ((0, 0), (0, DC - C)))
  ie2, y2 = _tc_fuse2(a1p, W1, b1.reshape(1, D), Wi1, bi1.reshape(1, C),
                      W2p, b2.reshape(1, C), ie0)
  a2p = mp64(srcp, dstp, zeros64, y2)
  return _tc_fuse3(a2p, ie2)


# speedup vs baseline: 1.8165x; 1.0361x over previous
import functools

import jax
import jax.numpy as jnp
from jax import lax
from jax.experimental import pallas as pl
from jax.experimental.pallas import tpu as pltpu
from jax.experimental.pallas import tpu_sc as plsc

N = 10000
D = 128
DC = 48
C = 40
NSC = 2
NTILES = 16
EDGE_BLOCKS = NSC * NTILES
E = 320000
CHUNK = 200
CA = 51
CB = 49
CHUNK3 = 400
CA3 = 25
CB3 = 25
ACC_ROWS = N + 16
ZROWS = ACC_ROWS // NTILES
OROWS = N // NTILES


def _make_msgpass(d, chunk, ca, cb):
  mesh = plsc.VectorSubcoreMesh(core_axis_name="c", subcore_axis_name="s")
  cmax = max(ca, cb)

  @functools.partial(
      pl.kernel,
      out_type=jax.ShapeDtypeStruct((NSC, N, d), jnp.float32),
      mesh=mesh,
      scratch_types=[
          pltpu.VMEM((cmax, chunk), jnp.int32),
          pltpu.VMEM((cmax, chunk), jnp.int32),
          pltpu.VMEM((chunk, d), jnp.float32),
          pltpu.VMEM_SHARED((ACC_ROWS, d), jnp.float32),
          pltpu.SemaphoreType.DMA,
      ],
      compiler_params=pltpu.CompilerParams(use_tc_tiling_on_sc=False),
  )
  def msgpass(src_hbm, dst_hbm, zeros_hbm, y_hbm, out_hbm,
              idx_s, idx_d, rows, acc, sem):
    c = lax.axis_index("c")
    s = lax.axis_index("s")

    @pl.when(c == 0)
    def _():
      pltpu.sync_copy(src_hbm.at[pl.ds(s * ca, ca)], idx_s.at[pl.ds(0, ca)])
      pltpu.sync_copy(dst_hbm.at[pl.ds(s * ca, ca)], idx_d.at[pl.ds(0, ca)])

    @pl.when(c == 1)
    def _():
      pltpu.sync_copy(src_hbm.at[pl.ds(NTILES * ca + s * cb, cb)],
                      idx_s.at[pl.ds(0, cb)])
      pltpu.sync_copy(dst_hbm.at[pl.ds(NTILES * ca + s * cb, cb)],
                      idx_d.at[pl.ds(0, cb)])

    pltpu.sync_copy(zeros_hbm, acc.at[pl.ds(s * ZROWS, ZROWS)])
    plsc.subcore_barrier()
    n = jnp.where(c == 0, ca, cb)

    @pl.loop(0, n)
    def _(j):
      pltpu.async_copy(y_hbm.at[idx_s.at[j]], rows, sem).wait()
      pltpu.sync_copy(rows, acc.at[idx_d.at[j]], add=True)

    plsc.subcore_barrier()
    pltpu.sync_copy(acc.at[pl.ds(s * OROWS, OROWS)],
                    out_hbm.at[c, pl.ds(s * OROWS, OROWS)])

  return msgpass


_R = 1000


def _tc_fuse1(a0p, W0, b0, Wi0, bi0):
  def body(a_ref, w_ref, b_ref, wi_ref, bi_ref, h_ref, ie_ref):
    a = a_ref[0] + a_ref[1]
    h = jnp.maximum(
        jnp.dot(a, w_ref[...], preferred_element_type=jnp.float32) + b_ref[...],
        0.0)
    h_ref[...] = h
    ie_ref[...] = (
        jnp.dot(h, wi_ref[...], preferred_element_type=jnp.float32) + bi_ref[...])

  return pl.pallas_call(
      body,
      grid=(N // _R,),
      in_specs=[
          pl.BlockSpec((NSC, _R, D), lambda i: (0, i, 0)),
          pl.BlockSpec((D, D), lambda i: (0, 0)),
          pl.BlockSpec((1, D), lambda i: (0, 0)),
          pl.BlockSpec((D, C), lambda i: (0, 0)),
          pl.BlockSpec((1, C), lambda i: (0, 0)),
      ],
      out_specs=[
          pl.BlockSpec((_R, D), lambda i: (i, 0)),
          pl.BlockSpec((_R, C), lambda i: (i, 0)),
      ],
      out_shape=[
          jax.ShapeDtypeStruct((N, D), jnp.float32),
          jax.ShapeDtypeStruct((N, C), jnp.float32),
      ],
  )(a0p, W0, b0, Wi0, bi0)


def _tc_fuse2(a1p, W1, b1, Wi1, bi1, W2p, b2, ie0):
  def body(a_ref, w_ref, b_ref, wi_ref, bi_ref, w2_ref, b2_ref, ie_ref,
           ie2_ref, y2_ref):
    a = a_ref[0] + a_ref[1]
    h = jnp.maximum(
        jnp.dot(a, w_ref[...], preferred_element_type=jnp.float32) + b_ref[...],
        0.0)
    ie2_ref[...] = (
        ie_ref[...]
        + jnp.dot(h, wi_ref[...], preferred_element_type=jnp.float32)
        + bi_ref[...] + b2_ref[...])
    y2_ref[...] = jnp.dot(h, w2_ref[...], preferred_element_type=jnp.float32)

  return pl.pallas_call(
      body,
      grid=(N // _R,),
      in_specs=[
          pl.BlockSpec((NSC, _R, D), lambda i: (0, i, 0)),
          pl.BlockSpec((D, D), lambda i: (0, 0)),
          pl.BlockSpec((1, D), lambda i: (0, 0)),
          pl.BlockSpec((D, C), lambda i: (0, 0)),
          pl.BlockSpec((1, C), lambda i: (0, 0)),
          pl.BlockSpec((D, DC), lambda i: (0, 0)),
          pl.BlockSpec((1, C), lambda i: (0, 0)),
          pl.BlockSpec((_R, C), lambda i: (i, 0)),
      ],
      out_specs=[
          pl.BlockSpec((_R, C), lambda i: (i, 0)),
          pl.BlockSpec((_R, DC), lambda i: (i, 0)),
      ],
      out_shape=[
          jax.ShapeDtypeStruct((N, C), jnp.float32),
          jax.ShapeDtypeStruct((N, DC), jnp.float32),
      ],
  )(a1p, W1, b1, Wi1, bi1, W2p, b2, ie0)


def _tc_fuse3(a2p, ie2):
  def body(a_ref, ie_ref, o_ref):
    o_ref[...] = a_ref[0][:, :C] + a_ref[1][:, :C] + ie_ref[...]

  return pl.pallas_call(
      body,
      grid=(N // _R,),
      in_specs=[
          pl.BlockSpec((NSC, _R, DC), lambda i: (0, i, 0)),
          pl.BlockSpec((_R, C), lambda i: (i, 0)),
      ],
      out_specs=pl.BlockSpec((_R, C), lambda i: (i, 0)),
      out_shape=jax.ShapeDtypeStruct((N, C), jnp.float32),
  )(a2p, ie2)


def kernel(graph, features, W0, b0, W1, b1, W2, b2, Wi0, bi0, Wi1, bi1):
  src, dst = graph[0], graph[1]
  srcp = src.reshape(E // CHUNK, CHUNK)
  dstp = dst.reshape(E // CHUNK, CHUNK)
  srcp3 = src.reshape(E // CHUNK3, CHUNK3)
  dstp3 = dst.reshape(E // CHUNK3, CHUNK3)
  zeros128 = jnp.zeros((ZROWS, D), jnp.float32)
  zeros64 = jnp.zeros((ZROWS, DC), jnp.float32)

  mp128 = _make_msgpass(D, CHUNK, CA, CB)
  mp48 = _make_msgpass(DC, CHUNK3, CA3, CB3)

  a0p = mp128(srcp, dstp, zeros128, features)
  h1, ie0 = _tc_fuse1(a0p, W0, b0.reshape(1, D), Wi0, bi0.reshape(1, C))
  a1p = mp128(srcp, dstp, zeros128, h1)
  W2p = jnp.pad(W2, ((0, 0), (0, DC - C)))
  ie2, y2 = _tc_fuse2(a1p, W1, b1.reshape(1, D), Wi1, bi1.reshape(1, C),
                      W2p, b2.reshape(1, C), ie0)
  a2p = mp48(srcp3, dstp3, zeros64, y2)
  return _tc_fuse3(a2p, ie2)


# speedup vs baseline: 1.8618x; 1.0250x over previous
import functools

import jax
import jax.numpy as jnp
from jax import lax
from jax.experimental import pallas as pl
from jax.experimental.pallas import tpu as pltpu
from jax.experimental.pallas import tpu_sc as plsc

N = 10000
D = 128
DC = 48
C = 40
NSC = 2
NTILES = 16
EDGE_BLOCKS = NSC * NTILES
E = 320000
CHUNK = 200
CA = 51
CB = 49
CHUNK3 = 1000
CA3 = 10
CB3 = 10
ACC_ROWS = N
ZROWS = ACC_ROWS // NTILES
OROWS = N // NTILES


def _make_msgpass(d, chunk, ca, cb):
  mesh = plsc.VectorSubcoreMesh(core_axis_name="c", subcore_axis_name="s")
  cmax = max(ca, cb)

  @functools.partial(
      pl.kernel,
      out_type=jax.ShapeDtypeStruct((NSC, N, d), jnp.float32),
      mesh=mesh,
      scratch_types=[
          pltpu.VMEM((cmax, chunk), jnp.int32),
          pltpu.VMEM((cmax, chunk), jnp.int32),
          pltpu.VMEM((chunk, d), jnp.float32),
          pltpu.VMEM_SHARED((ACC_ROWS, d), jnp.float32),
          pltpu.SemaphoreType.DMA,
      ],
      compiler_params=pltpu.CompilerParams(use_tc_tiling_on_sc=False),
  )
  def msgpass(src_hbm, dst_hbm, zeros_hbm, y_hbm, out_hbm,
              idx_s, idx_d, rows, acc, sem):
    c = lax.axis_index("c")
    s = lax.axis_index("s")

    @pl.when(c == 0)
    def _():
      pltpu.async_copy(src_hbm.at[pl.ds(s * ca, ca)], idx_s.at[pl.ds(0, ca)],
                       sem)
      pltpu.async_copy(dst_hbm.at[pl.ds(s * ca, ca)], idx_d.at[pl.ds(0, ca)],
                       sem)

    @pl.when(c == 1)
    def _():
      pltpu.async_copy(src_hbm.at[pl.ds(NTILES * ca + s * cb, cb)],
                       idx_s.at[pl.ds(0, cb)], sem)
      pltpu.async_copy(dst_hbm.at[pl.ds(NTILES * ca + s * cb, cb)],
                       idx_d.at[pl.ds(0, cb)], sem)

    pltpu.async_copy(zeros_hbm, acc.at[pl.ds(s * ZROWS, ZROWS)], sem)

    @pl.when(c == 0)
    def _():
      pltpu.make_async_copy(src_hbm.at[pl.ds(s * ca, ca)],
                            idx_s.at[pl.ds(0, ca)], sem).wait()
      pltpu.make_async_copy(dst_hbm.at[pl.ds(s * ca, ca)],
                            idx_d.at[pl.ds(0, ca)], sem).wait()

    @pl.when(c == 1)
    def _():
      pltpu.make_async_copy(src_hbm.at[pl.ds(NTILES * ca + s * cb, cb)],
                            idx_s.at[pl.ds(0, cb)], sem).wait()
      pltpu.make_async_copy(dst_hbm.at[pl.ds(NTILES * ca + s * cb, cb)],
                            idx_d.at[pl.ds(0, cb)], sem).wait()

    pltpu.make_async_copy(zeros_hbm, acc.at[pl.ds(s * ZROWS, ZROWS)],
                          sem).wait()
    plsc.subcore_barrier()
    n = jnp.where(c == 0, ca, cb)

    @pl.loop(0, n)
    def _(j):
      pltpu.async_copy(y_hbm.at[idx_s.at[j]], rows, sem).wait()
      pltpu.sync_copy(rows, acc.at[idx_d.at[j]], add=True)

    plsc.subcore_barrier()
    pltpu.sync_copy(acc.at[pl.ds(s * OROWS, OROWS)],
                    out_hbm.at[c, pl.ds(s * OROWS, OROWS)])

  return msgpass


_R = 1000


def _tc_fuse1(a0p, W0, b0, Wi0, bi0):
  def body(a_ref, w_ref, b_ref, wi_ref, bi_ref, h_ref, ie_ref):
    a = a_ref[0] + a_ref[1]
    h = jnp.maximum(
        jnp.dot(a, w_ref[...], preferred_element_type=jnp.float32) + b_ref[...],
        0.0)
    h_ref[...] = h
    ie_ref[...] = (
        jnp.dot(h, wi_ref[...], preferred_element_type=jnp.float32) + bi_ref[...])

  return pl.pallas_call(
      body,
      grid=(N // _R,),
      in_specs=[
          pl.BlockSpec((NSC, _R, D), lambda i: (0, i, 0)),
          pl.BlockSpec((D, D), lambda i: (0, 0)),
          pl.BlockSpec((1, D), lambda i: (0, 0)),
          pl.BlockSpec((D, C), lambda i: (0, 0)),
          pl.BlockSpec((1, C), lambda i: (0, 0)),
      ],
      out_specs=[
          pl.BlockSpec((_R, D), lambda i: (i, 0)),
          pl.BlockSpec((_R, C), lambda i: (i, 0)),
      ],
      out_shape=[
          jax.ShapeDtypeStruct((N, D), jnp.float32),
          jax.ShapeDtypeStruct((N, C), jnp.float32),
      ],
  )(a0p, W0, b0, Wi0, bi0)


def _tc_fuse2(a1p, W1, b1, Wi1, bi1, W2p, b2, ie0):
  def body(a_ref, w_ref, b_ref, wi_ref, bi_ref, w2_ref, b2_ref, ie_ref,
           ie2_ref, y2_ref):
    a = a_ref[0] + a_ref[1]
    h = jnp.maximum(
        jnp.dot(a, w_ref[...], preferred_element_type=jnp.float32) + b_ref[...],
        0.0)
    ie2_ref[...] = (
        ie_ref[...]
        + jnp.dot(h, wi_ref[...], preferred_element_type=jnp.float32)
        + bi_ref[...] + b2_ref[...])
    y2_ref[...] = jnp.dot(h, w2_ref[...], preferred_element_type=jnp.float32)

  return pl.pallas_call(
      body,
      grid=(N // _R,),
      in_specs=[
          pl.BlockSpec((NSC, _R, D), lambda i: (0, i, 0)),
          pl.BlockSpec((D, D), lambda i: (0, 0)),
          pl.BlockSpec((1, D), lambda i: (0, 0)),
          pl.BlockSpec((D, C), lambda i: (0, 0)),
          pl.BlockSpec((1, C), lambda i: (0, 0)),
          pl.BlockSpec((D, DC), lambda i: (0, 0)),
          pl.BlockSpec((1, C), lambda i: (0, 0)),
          pl.BlockSpec((_R, C), lambda i: (i, 0)),
      ],
      out_specs=[
          pl.BlockSpec((_R, C), lambda i: (i, 0)),
          pl.BlockSpec((_R, DC), lambda i: (i, 0)),
      ],
      out_shape=[
          jax.ShapeDtypeStruct((N, C), jnp.float32),
          jax.ShapeDtypeStruct((N, DC), jnp.float32),
      ],
  )(a1p, W1, b1, Wi1, bi1, W2p, b2, ie0)


def _tc_fuse3(a2p, ie2):
  def body(a_ref, ie_ref, o_ref):
    o_ref[...] = a_ref[0][:, :C] + a_ref[1][:, :C] + ie_ref[...]

  return pl.pallas_call(
      body,
      grid=(N // _R,),
      in_specs=[
          pl.BlockSpec((NSC, _R, DC), lambda i: (0, i, 0)),
          pl.BlockSpec((_R, C), lambda i: (i, 0)),
      ],
      out_specs=pl.BlockSpec((_R, C), lambda i: (i, 0)),
      out_shape=jax.ShapeDtypeStruct((N, C), jnp.float32),
  )(a2p, ie2)


def kernel(graph, features, W0, b0, W1, b1, W2, b2, Wi0, bi0, Wi1, bi1):
  src, dst = graph[0], graph[1]
  srcp = src.reshape(E // CHUNK, CHUNK)
  dstp = dst.reshape(E // CHUNK, CHUNK)
  srcp3 = src.reshape(E // CHUNK3, CHUNK3)
  dstp3 = dst.reshape(E // CHUNK3, CHUNK3)
  zeros128 = jnp.zeros((ZROWS, D), jnp.float32)
  zeros64 = jnp.zeros((ZROWS, DC), jnp.float32)

  mp128 = _make_msgpass(D, CHUNK, CA, CB)
  mp48 = _make_msgpass(DC, CHUNK3, CA3, CB3)

  a0p = mp128(srcp, dstp, zeros128, features)
  h1, ie0 = _tc_fuse1(a0p, W0, b0.reshape(1, D), Wi0, bi0.reshape(1, C))
  a1p = mp128(srcp, dstp, zeros128, h1)
  W2p = jnp.pad(W2, ((0, 0), (0, DC - C)))
  ie2, y2 = _tc_fuse2(a1p, W1, b1.reshape(1, D), Wi1, bi1.reshape(1, C),
                      W2p, b2.reshape(1, C), ie0)
  a2p = mp48(srcp3, dstp3, zeros64, y2)
  return _tc_fuse3(a2p, ie2)


# speedup vs baseline: 1.9024x; 1.0218x over previous
import functools

import jax
import jax.numpy as jnp
from jax import lax
from jax.experimental import pallas as pl
from jax.experimental.pallas import tpu as pltpu
from jax.experimental.pallas import tpu_sc as plsc

N = 10000
D = 128
DC = 48
C = 40
NSC = 2
NTILES = 16
EDGE_BLOCKS = NSC * NTILES
E = 320000
CHUNK = 200
CA = 50
CB = 50
CHUNK3 = 1000
CA3 = 10
CB3 = 10
ACC_ROWS = N
ZROWS = ACC_ROWS // NTILES
OROWS = N // NTILES


def _make_msgpass(d, chunk, ca, cb, stage_table=False):
  mesh = plsc.VectorSubcoreMesh(core_axis_name="c", subcore_axis_name="s")
  cmax = max(ca, cb)

  @functools.partial(
      pl.kernel,
      out_type=jax.ShapeDtypeStruct((NSC, N, d), jnp.float32),
      mesh=mesh,
      scratch_types=[
          pltpu.VMEM((cmax, chunk), jnp.int32),
          pltpu.VMEM((cmax, chunk), jnp.int32),
          pltpu.VMEM((chunk, d), jnp.float32),
          pltpu.VMEM_SHARED((ACC_ROWS, d), jnp.float32),
          pltpu.VMEM_SHARED((N, d), jnp.float32) if stage_table else None,
          pltpu.SemaphoreType.DMA,
      ],
      compiler_params=pltpu.CompilerParams(use_tc_tiling_on_sc=False),
  )
  def msgpass(src_hbm, dst_hbm, zeros_hbm, y_hbm, out_hbm,
              idx_s, idx_d, rows, acc, table, sem):
    c = lax.axis_index("c")
    s = lax.axis_index("s")

    @pl.when(c == 0)
    def _():
      pltpu.async_copy(src_hbm.at[pl.ds(s * ca, ca)], idx_s.at[pl.ds(0, ca)],
                       sem)
      pltpu.async_copy(dst_hbm.at[pl.ds(s * ca, ca)], idx_d.at[pl.ds(0, ca)],
                       sem)

    @pl.when(c == 1)
    def _():
      pltpu.async_copy(src_hbm.at[pl.ds(NTILES * ca + s * cb, cb)],
                       idx_s.at[pl.ds(0, cb)], sem)
      pltpu.async_copy(dst_hbm.at[pl.ds(NTILES * ca + s * cb, cb)],
                       idx_d.at[pl.ds(0, cb)], sem)

    pltpu.async_copy(zeros_hbm, acc.at[pl.ds(s * ZROWS, ZROWS)], sem)
    if stage_table:
      pltpu.async_copy(y_hbm.at[pl.ds(s * OROWS, OROWS)],
                       table.at[pl.ds(s * OROWS, OROWS)], sem)

    @pl.when(c == 0)
    def _():
      pltpu.make_async_copy(src_hbm.at[pl.ds(s * ca, ca)],
                            idx_s.at[pl.ds(0, ca)], sem).wait()
      pltpu.make_async_copy(dst_hbm.at[pl.ds(s * ca, ca)],
                            idx_d.at[pl.ds(0, ca)], sem).wait()

    @pl.when(c == 1)
    def _():
      pltpu.make_async_copy(src_hbm.at[pl.ds(NTILES * ca + s * cb, cb)],
                            idx_s.at[pl.ds(0, cb)], sem).wait()
      pltpu.make_async_copy(dst_hbm.at[pl.ds(NTILES * ca + s * cb, cb)],
                            idx_d.at[pl.ds(0, cb)], sem).wait()

    pltpu.make_async_copy(zeros_hbm, acc.at[pl.ds(s * ZROWS, ZROWS)],
                          sem).wait()
    if stage_table:
      pltpu.make_async_copy(y_hbm.at[pl.ds(s * OROWS, OROWS)],
                            table.at[pl.ds(s * OROWS, OROWS)], sem).wait()
    plsc.subcore_barrier()
    n = jnp.where(c == 0, ca, cb)

    gsrc = table if stage_table else y_hbm

    @pl.loop(0, n)
    def _(j):
      pltpu.async_copy(gsrc.at[idx_s.at[j]], rows, sem).wait()
      pltpu.sync_copy(rows, acc.at[idx_d.at[j]], add=True)

    plsc.subcore_barrier()
    pltpu.sync_copy(acc.at[pl.ds(s * OROWS, OROWS)],
                    out_hbm.at[c, pl.ds(s * OROWS, OROWS)])

  return msgpass


_R = 1000


def _tc_fuse1(a0p, W0, b0, Wi0, bi0):
  def body(a_ref, w_ref, b_ref, wi_ref, bi_ref, h_ref, ie_ref):
    a = a_ref[0] + a_ref[1]
    h = jnp.maximum(
        jnp.dot(a, w_ref[...], preferred_element_type=jnp.float32) + b_ref[...],
        0.0)
    h_ref[...] = h
    ie_ref[...] = (
        jnp.dot(h, wi_ref[...], preferred_element_type=jnp.float32) + bi_ref[...])

  return pl.pallas_call(
      body,
      grid=(N // _R,),
      in_specs=[
          pl.BlockSpec((NSC, _R, D), lambda i: (0, i, 0)),
          pl.BlockSpec((D, D), lambda i: (0, 0)),
          pl.BlockSpec((1, D), lambda i: (0, 0)),
          pl.BlockSpec((D, C), lambda i: (0, 0)),
          pl.BlockSpec((1, C), lambda i: (0, 0)),
      ],
      out_specs=[
          pl.BlockSpec((_R, D), lambda i: (i, 0)),
          pl.BlockSpec((_R, C), lambda i: (i, 0)),
      ],
      out_shape=[
          jax.ShapeDtypeStruct((N, D), jnp.float32),
          jax.ShapeDtypeStruct((N, C), jnp.float32),
      ],
  )(a0p, W0, b0, Wi0, bi0)


def _tc_fuse2(a1p, W1, b1, Wi1, bi1, W2p, b2, ie0):
  def body(a_ref, w_ref, b_ref, wi_ref, bi_ref, w2_ref, b2_ref, ie_ref,
           ie2_ref, y2_ref):
    a = a_ref[0] + a_ref[1]
    h = jnp.maximum(
        jnp.dot(a, w_ref[...], preferred_element_type=jnp.float32) + b_ref[...],
        0.0)
    ie2_ref[...] = (
        ie_ref[...]
        + jnp.dot(h, wi_ref[...], preferred_element_type=jnp.float32)
        + bi_ref[...] + b2_ref[...])
    y2_ref[...] = jnp.dot(h, w2_ref[...], preferred_element_type=jnp.float32)

  return pl.pallas_call(
      body,
      grid=(N // _R,),
      in_specs=[
          pl.BlockSpec((NSC, _R, D), lambda i: (0, i, 0)),
          pl.BlockSpec((D, D), lambda i: (0, 0)),
          pl.BlockSpec((1, D), lambda i: (0, 0)),
          pl.BlockSpec((D, C), lambda i: (0, 0)),
          pl.BlockSpec((1, C), lambda i: (0, 0)),
          pl.BlockSpec((D, DC), lambda i: (0, 0)),
          pl.BlockSpec((1, C), lambda i: (0, 0)),
          pl.BlockSpec((_R, C), lambda i: (i, 0)),
      ],
      out_specs=[
          pl.BlockSpec((_R, C), lambda i: (i, 0)),
          pl.BlockSpec((_R, DC), lambda i: (i, 0)),
      ],
      out_shape=[
          jax.ShapeDtypeStruct((N, C), jnp.float32),
          jax.ShapeDtypeStruct((N, DC), jnp.float32),
      ],
  )(a1p, W1, b1, Wi1, bi1, W2p, b2, ie0)


def _tc_fuse3(a2p, ie2):
  def body(a_ref, ie_ref, o_ref):
    o_ref[...] = a_ref[0][:, :C] + a_ref[1][:, :C] + ie_ref[...]

  return pl.pallas_call(
      body,
      grid=(N // _R,),
      in_specs=[
          pl.BlockSpec((NSC, _R, DC), lambda i: (0, i, 0)),
          pl.BlockSpec((_R, C), lambda i: (i, 0)),
      ],
      out_specs=pl.BlockSpec((_R, C), lambda i: (i, 0)),
      out_shape=jax.ShapeDtypeStruct((N, C), jnp.float32),
  )(a2p, ie2)


def kernel(graph, features, W0, b0, W1, b1, W2, b2, Wi0, bi0, Wi1, bi1):
  src, dst = graph[0], graph[1]
  srcp = src.reshape(E // CHUNK, CHUNK)
  dstp = dst.reshape(E // CHUNK, CHUNK)
  srcp3 = src.reshape(E // CHUNK3, CHUNK3)
  dstp3 = dst.reshape(E // CHUNK3, CHUNK3)
  zeros128 = jnp.zeros((ZROWS, D), jnp.float32)
  zeros64 = jnp.zeros((ZROWS, DC), jnp.float32)

  mp128 = _make_msgpass(D, CHUNK, CA, CB)
  mp48 = _make_msgpass(DC, CHUNK3, CA3, CB3, stage_table=True)

  a0p = mp128(srcp, dstp, zeros128, features)
  h1, ie0 = _tc_fuse1(a0p, W0, b0.reshape(1, D), Wi0, bi0.reshape(1, C))
  a1p = mp128(srcp, dstp, zeros128, h1)
  W2p = jnp.pad(W2, ((0, 0), (0, DC - C)))
  ie2, y2 = _tc_fuse2(a1p, W1, b1.reshape(1, D), Wi1, bi1.reshape(1, C),
                      W2p, b2.reshape(1, C), ie0)
  a2p = mp48(srcp3, dstp3, zeros64, y2)
  return _tc_fuse3(a2p, ie2)


# speedup vs baseline: 1.9065x; 1.0021x over previous
import functools

import jax
import jax.numpy as jnp
from jax import lax
from jax.experimental import pallas as pl
from jax.experimental.pallas import tpu as pltpu
from jax.experimental.pallas import tpu_sc as plsc

N = 10000
D = 128
DC = 48
C = 40
NSC = 2
NTILES = 16
E = 320000
CHUNK = 200
CA = 50
CB = 50
CHUNK3 = 1000
CA3 = 10
CB3 = 10
ACC_ROWS = N
ZROWS = ACC_ROWS // NTILES
OROWS = N // NTILES


def _make_msgpass(d, chunk, ca, cb, stage_table=False):
  mesh = plsc.VectorSubcoreMesh(core_axis_name="c", subcore_axis_name="s")
  cmax = max(ca, cb)

  @functools.partial(
      pl.kernel,
      out_type=jax.ShapeDtypeStruct((NSC, N, d), jnp.float32),
      mesh=mesh,
      scratch_types=[
          pltpu.VMEM((cmax, chunk), jnp.int32),
          pltpu.VMEM((cmax, chunk), jnp.int32),
          pltpu.VMEM((chunk, d), jnp.float32),
          pltpu.VMEM_SHARED((ACC_ROWS, d), jnp.float32),
          pltpu.VMEM_SHARED((N, d), jnp.float32) if stage_table else None,
          pltpu.SemaphoreType.DMA,
      ],
      compiler_params=pltpu.CompilerParams(use_tc_tiling_on_sc=False),
  )
  def msgpass(src_hbm, dst_hbm, zeros_hbm, y_hbm, out_hbm,
              idx_s, idx_d, rows, acc, table, sem):
    c = lax.axis_index("c")
    s = lax.axis_index("s")

    @pl.when(c == 0)
    def _():
      pltpu.async_copy(src_hbm.at[pl.ds(s * ca, ca)], idx_s.at[pl.ds(0, ca)],
                       sem)
      pltpu.async_copy(dst_hbm.at[pl.ds(s * ca, ca)], idx_d.at[pl.ds(0, ca)],
                       sem)

    @pl.when(c == 1)
    def _():
      pltpu.async_copy(src_hbm.at[pl.ds(NTILES * ca + s * cb, cb)],
                       idx_s.at[pl.ds(0, cb)], sem)
      pltpu.async_copy(dst_hbm.at[pl.ds(NTILES * ca + s * cb, cb)],
                       idx_d.at[pl.ds(0, cb)], sem)

    pltpu.async_copy(zeros_hbm, acc.at[pl.ds(s * ZROWS, ZROWS)], sem)
    if stage_table:
      pltpu.async_copy(y_hbm.at[pl.ds(s * OROWS, OROWS)],
                       table.at[pl.ds(s * OROWS, OROWS)], sem)

    @pl.when(c == 0)
    def _():
      pltpu.make_async_copy(src_hbm.at[pl.ds(s * ca, ca)],
                            idx_s.at[pl.ds(0, ca)], sem).wait()
      pltpu.make_async_copy(dst_hbm.at[pl.ds(s * ca, ca)],
                            idx_d.at[pl.ds(0, ca)], sem).wait()

    @pl.when(c == 1)
    def _():
      pltpu.make_async_copy(src_hbm.at[pl.ds(NTILES * ca + s * cb, cb)],
                            idx_s.at[pl.ds(0, cb)], sem).wait()
      pltpu.make_async_copy(dst_hbm.at[pl.ds(NTILES * ca + s * cb, cb)],
                            idx_d.at[pl.ds(0, cb)], sem).wait()

    pltpu.make_async_copy(zeros_hbm, acc.at[pl.ds(s * ZROWS, ZROWS)],
                          sem).wait()
    if stage_table:
      pltpu.make_async_copy(y_hbm.at[pl.ds(s * OROWS, OROWS)],
                            table.at[pl.ds(s * OROWS, OROWS)], sem).wait()
    plsc.subcore_barrier()
    n = jnp.where(c == 0, ca, cb)

    gsrc = table if stage_table else y_hbm

    @pl.loop(0, n)
    def _(j):
      pltpu.async_copy(gsrc.at[idx_s.at[j]], rows, sem).wait()
      pltpu.sync_copy(rows, acc.at[idx_d.at[j]], add=True)

    plsc.subcore_barrier()
    pltpu.sync_copy(acc.at[pl.ds(s * OROWS, OROWS)],
                    out_hbm.at[c, pl.ds(s * OROWS, OROWS)])

  return msgpass


_R = 1000


def _tc_fuse1(a0p, W0, b0, Wi0, bi0):
  def body(a_ref, w_ref, b_ref, wi_ref, bi_ref, h_ref, ie_ref):
    a = a_ref[0] + a_ref[1]
    h = jnp.maximum(
        jnp.dot(a, w_ref[...], preferred_element_type=jnp.float32) + b_ref[...],
        0.0)
    h_ref[...] = h
    ie_ref[...] = (
        jnp.dot(h, wi_ref[...], preferred_element_type=jnp.float32) + bi_ref[...])

  return pl.pallas_call(
      body,
      grid=(N // _R,),
      in_specs=[
          pl.BlockSpec((NSC, _R, D), lambda i: (0, i, 0)),
          pl.BlockSpec((D, D), lambda i: (0, 0)),
          pl.BlockSpec((1, D), lambda i: (0, 0)),
          pl.BlockSpec((D, C), lambda i: (0, 0)),
          pl.BlockSpec((1, C), lambda i: (0, 0)),
      ],
      out_specs=[
          pl.BlockSpec((_R, D), lambda i: (i, 0)),
          pl.BlockSpec((_R, C), lambda i: (i, 0)),
      ],
      out_shape=[
          jax.ShapeDtypeStruct((N, D), jnp.float32),
          jax.ShapeDtypeStruct((N, C), jnp.float32),
      ],
  )(a0p, W0, b0, Wi0, bi0)


def _tc_fuse2(a1p, W1, b1, Wi1, bi1, W2p, b2, ie0):
  def body(a_ref, w_ref, b_ref, wi_ref, bi_ref, w2_ref, b2_ref, ie_ref,
           ie2_ref, y2_ref):
    a = a_ref[0] + a_ref[1]
    h = jnp.maximum(
        jnp.dot(a, w_ref[...], preferred_element_type=jnp.float32) + b_ref[...],
        0.0)
    ie2_ref[...] = (
        ie_ref[...]
        + jnp.dot(h, wi_ref[...], preferred_element_type=jnp.float32)
        + bi_ref[...] + b2_ref[...])
    y2_ref[...] = jnp.dot(h, w2_ref[...], preferred_element_type=jnp.float32)

  return pl.pallas_call(
      body,
      grid=(N // _R,),
      in_specs=[
          pl.BlockSpec((NSC, _R, D), lambda i: (0, i, 0)),
          pl.BlockSpec((D, D), lambda i: (0, 0)),
          pl.BlockSpec((1, D), lambda i: (0, 0)),
          pl.BlockSpec((D, C), lambda i: (0, 0)),
          pl.BlockSpec((1, C), lambda i: (0, 0)),
          pl.BlockSpec((D, DC), lambda i: (0, 0)),
          pl.BlockSpec((1, C), lambda i: (0, 0)),
          pl.BlockSpec((_R, C), lambda i: (i, 0)),
      ],
      out_specs=[
          pl.BlockSpec((_R, C), lambda i: (i, 0)),
          pl.BlockSpec((_R, DC), lambda i: (i, 0)),
      ],
      out_shape=[
          jax.ShapeDtypeStruct((N, C), jnp.float32),
          jax.ShapeDtypeStruct((N, DC), jnp.float32),
      ],
  )(a1p, W1, b1, Wi1, bi1, W2p, b2, ie0)


def _tc_fuse3(a2p, ie2):
  def body(a_ref, ie_ref, o_ref):
    o_ref[...] = a_ref[0][:, :C] + a_ref[1][:, :C] + ie_ref[...]

  return pl.pallas_call(
      body,
      grid=(N // _R,),
      in_specs=[
          pl.BlockSpec((NSC, _R, DC), lambda i: (0, i, 0)),
          pl.BlockSpec((_R, C), lambda i: (i, 0)),
      ],
      out_specs=pl.BlockSpec((_R, C), lambda i: (i, 0)),
      out_shape=jax.ShapeDtypeStruct((N, C), jnp.float32),
  )(a2p, ie2)


def kernel(graph, features, W0, b0, W1, b1, W2, b2, Wi0, bi0, Wi1, bi1):
  src, dst = graph[0], graph[1]
  srcp = src.reshape(E // CHUNK, CHUNK)
  dstp = dst.reshape(E // CHUNK, CHUNK)
  srcp3 = src.reshape(E // CHUNK3, CHUNK3)
  dstp3 = dst.reshape(E // CHUNK3, CHUNK3)
  zeros128 = jnp.zeros((ZROWS, D), jnp.float32)
  zeros48 = jnp.zeros((ZROWS, DC), jnp.float32)

  mp128 = _make_msgpass(D, CHUNK, CA, CB)
  mp48 = _make_msgpass(DC, CHUNK3, CA3, CB3, stage_table=True)

  a0p = mp128(srcp, dstp, zeros128, features)
  h1, ie0 = _tc_fuse1(a0p, W0, b0.reshape(1, D), Wi0, bi0.reshape(1, C))
  a1p = mp128(srcp, dstp, zeros128, h1)
  W2p = jnp.pad(W2, ((0, 0), (0, DC - C)))
  ie2, y2 = _tc_fuse2(a1p, W1, b1.reshape(1, D), Wi1, bi1.reshape(1, C),
                      W2p, b2.reshape(1, C), ie0)
  a2p = mp48(srcp3, dstp3, zeros48, y2)
  return _tc_fuse3(a2p, ie2)
